# baseline (speedup 1.0000x reference)
import functools

import jax
import jax.numpy as jnp
from jax import lax
from jax.experimental import pallas as pl
from jax.experimental.pallas import tpu as pltpu
from jax.experimental.pallas import tpu_sc as plsc

_B, _T, _D, _F = 8, 2048, 512, 2048
_TS = 128
_NSC = 2
_NTEC = 16
_GCH = 64


def _fused_body(cc_ref, x_ref, w1_ref, b1_ref, w2_ref, b2_ref, gm_ref,
                bt_ref, o_ref, bidsg_ref):
    cc = cc_ref[0]
    T = cc.shape[0]
    D = x_ref.shape[2]
    v = cc
    k = 1
    while k < T:
        z = jnp.zeros((k, 1), jnp.int32)
        v = v + jnp.concatenate([z, v[:-k]], axis=0)
        k *= 2
    bids = v - v[0:1]
    b = pl.program_id(0)
    bidsg_ref[0] = bids + b * T
    nseg = bids[T - 1, 0] + 1

    x = x_ref[0]
    xh = x.astype(jnp.bfloat16)
    ones_col = jnp.ones((T, 1), jnp.bfloat16)
    slot_iota = lax.broadcasted_iota(jnp.int32, (T, _TS), 1)
    cdims = (((0,), (0,)), ((), ()))

    for t in range(T // _TS):
        @pl.when(t * _TS < nseg)
        def _():
            oneh = (bids == slot_iota + t * _TS).astype(jnp.bfloat16)
            cnt = lax.dot_general(oneh, ones_col, cdims,
                                  preferred_element_type=jnp.float32)
            seg = lax.dot_general(oneh, xh, cdims,
                                  preferred_element_type=jnp.float32)
            xm = seg * (1.0 / jnp.maximum(cnt, 1.0))
            mh = xm.astype(jnp.bfloat16)
            p = jnp.dot(mh, w1_ref[...], preferred_element_type=jnp.float32)
            h1 = jnp.maximum(p + b1_ref[...], 0.0).astype(jnp.bfloat16)
            q = jnp.dot(h1, w2_ref[...], preferred_element_type=jnp.float32)
            acc = xm + q + b2_ref[...]
            mu = jnp.mean(acc, axis=-1, keepdims=True)
            d = acc - mu
            var = jnp.mean(d * d, axis=-1, keepdims=True)
            o_ref[0, pl.ds(t * _TS, _TS)] = (
                gm_ref[...] * d * lax.rsqrt(var + 1e-3) + bt_ref[...])


def _decompress_body(table_hbm, idxg_hbm, out_hbm, idx_v, rows_v, sem):
    c = lax.axis_index("c")
    w = lax.axis_index("s")
    wid = w * _NSC + c
    n_chunks = idxg_hbm.shape[0] // _GCH // (_NSC * _NTEC)
    for j in range(n_chunks):
        base = (wid * n_chunks + j) * _GCH
        pltpu.sync_copy(idxg_hbm.at[pl.ds(base, _GCH)], idx_v)
        pltpu.async_copy(table_hbm.at[idx_v], rows_v, sem).wait()
        pltpu.sync_copy(rows_v, out_hbm.at[pl.ds(base, _GCH)])


def kernel(hidden_states, chord_changes, W1, b1, W2, b2, gamma, beta):
    B, T, D = hidden_states.shape
    F = W1.shape[1]
    cc3 = chord_changes.reshape(B, T, 1)

    w1b = W1.astype(jnp.bfloat16)
    w2b = W2.astype(jnp.bfloat16)
    b1r = b1.reshape(1, F)
    b2r = b2.reshape(1, D)
    gmr = gamma.reshape(1, D)
    btr = beta.reshape(1, D)

    def compress_ffn(x_half, cc_half):
        Bh = x_half.shape[0]
        return pl.pallas_call(
            _fused_body,
            grid=(Bh,),
            in_specs=[
                pl.BlockSpec((1, T, 1), lambda b: (b, 0, 0)),
                pl.BlockSpec((1, T, D), lambda b: (b, 0, 0)),
                pl.BlockSpec((D, F), lambda b: (0, 0)),
                pl.BlockSpec((1, F), lambda b: (0, 0)),
                pl.BlockSpec((F, D), lambda b: (0, 0)),
                pl.BlockSpec((1, D), lambda b: (0, 0)),
                pl.BlockSpec((1, D), lambda b: (0, 0)),
                pl.BlockSpec((1, D), lambda b: (0, 0)),
            ],
            out_specs=[pl.BlockSpec((1, T, D), lambda b: (b, 0, 0)),
                       pl.BlockSpec((1, T, 1), lambda b: (b, 0, 0))],
            out_shape=[jax.ShapeDtypeStruct((Bh, T, D), jnp.float32),
                       jax.ShapeDtypeStruct((Bh, T, 1), jnp.int32)],
        )(cc_half, x_half, w1b, b1r, w2b, b2r, gmr, btr)

    mesh = plsc.VectorSubcoreMesh(core_axis_name="c", subcore_axis_name="s")

    def decompress(seg_half, bidsg_half):
        n = bidsg_half.shape[0] * bidsg_half.shape[1]
        dec = functools.partial(
            pl.kernel,
            out_type=jax.ShapeDtypeStruct((n, D), jnp.float32),
            mesh=mesh,
            scratch_types=[
                pltpu.VMEM((_GCH,), jnp.int32),
                pltpu.VMEM((_GCH, D), jnp.float32),
                pltpu.SemaphoreType.DMA,
            ],
        )(_decompress_body)
        return dec(seg_half.reshape(n, D), bidsg_half.reshape(n))

    seg, bg = compress_ffn(hidden_states, cc3)
    return decompress(seg, bg).reshape(B, T, D)


# speedup vs baseline: 1.1536x; 1.1536x over previous
import functools

import jax
import jax.numpy as jnp
from jax import lax
from jax.experimental import pallas as pl
from jax.experimental.pallas import tpu as pltpu
from jax.experimental.pallas import tpu_sc as plsc

_B, _T, _D, _F = 8, 2048, 512, 2048
_TS = 256
_NSC = 2
_NTEC = 16
_GCH = 64


def _fused_body(cc_ref, x_ref, w1_ref, b1_ref, w2_ref, b2_ref, gm_ref,
                bt_ref, o_ref, bidsg_ref):
    cc = cc_ref[0]
    T = cc.shape[0]
    D = x_ref.shape[2]
    v = cc
    k = 1
    while k < T:
        z = jnp.zeros((k, 1), jnp.int32)
        v = v + jnp.concatenate([z, v[:-k]], axis=0)
        k *= 2
    bids = v - v[0:1]
    b = pl.program_id(0)
    bidsg_ref[0] = bids + b * T
    nseg = bids[T - 1, 0] + 1

    x = x_ref[0]
    xh = x.astype(jnp.bfloat16)
    ones_col = jnp.ones((T, 1), jnp.bfloat16)
    slot_iota = lax.broadcasted_iota(jnp.int32, (T, _TS), 1)
    cdims = (((0,), (0,)), ((), ()))

    for t in range(T // _TS):
        @pl.when(t * _TS < nseg)
        def _():
            oneh = (bids == slot_iota + t * _TS).astype(jnp.bfloat16)
            cnt = lax.dot_general(oneh, ones_col, cdims,
                                  preferred_element_type=jnp.float32)
            seg = lax.dot_general(oneh, xh, cdims,
                                  preferred_element_type=jnp.float32)
            xm = seg * (1.0 / jnp.maximum(cnt, 1.0))
            mh = xm.astype(jnp.bfloat16)
            p = jnp.dot(mh, w1_ref[...], preferred_element_type=jnp.float32)
            h1 = jnp.maximum(p + b1_ref[...], 0.0).astype(jnp.bfloat16)
            q = jnp.dot(h1, w2_ref[...], preferred_element_type=jnp.float32)
            acc = xm + q + b2_ref[...]
            mu = jnp.mean(acc, axis=-1, keepdims=True)
            d = acc - mu
            var = jnp.mean(d * d, axis=-1, keepdims=True)
            o_ref[0, pl.ds(t * _TS, _TS)] = (
                gm_ref[...] * d * lax.rsqrt(var + 1e-3) + bt_ref[...])


def _decompress_body(table_hbm, idxg_hbm, out_hbm, idx_v, rows_v, sem):
    c = lax.axis_index("c")
    w = lax.axis_index("s")
    wid = w * _NSC + c
    n_chunks = idxg_hbm.shape[0] // _GCH // (_NSC * _NTEC)
    for j in range(n_chunks):
        base = (wid * n_chunks + j) * _GCH
        pltpu.sync_copy(idxg_hbm.at[pl.ds(base, _GCH)], idx_v)
        pltpu.async_copy(table_hbm.at[idx_v], rows_v, sem).wait()
        pltpu.sync_copy(rows_v, out_hbm.at[pl.ds(base, _GCH)])


def kernel(hidden_states, chord_changes, W1, b1, W2, b2, gamma, beta):
    B, T, D = hidden_states.shape
    F = W1.shape[1]
    cc3 = chord_changes.reshape(B, T, 1)

    w1b = W1.astype(jnp.bfloat16)
    w2b = W2.astype(jnp.bfloat16)
    b1r = b1.reshape(1, F)
    b2r = b2.reshape(1, D)
    gmr = gamma.reshape(1, D)
    btr = beta.reshape(1, D)

    def compress_ffn(x_half, cc_half):
        Bh = x_half.shape[0]
        return pl.pallas_call(
            _fused_body,
            grid=(Bh,),
            in_specs=[
                pl.BlockSpec((1, T, 1), lambda b: (b, 0, 0)),
                pl.BlockSpec((1, T, D), lambda b: (b, 0, 0)),
                pl.BlockSpec((D, F), lambda b: (0, 0)),
                pl.BlockSpec((1, F), lambda b: (0, 0)),
                pl.BlockSpec((F, D), lambda b: (0, 0)),
                pl.BlockSpec((1, D), lambda b: (0, 0)),
                pl.BlockSpec((1, D), lambda b: (0, 0)),
                pl.BlockSpec((1, D), lambda b: (0, 0)),
            ],
            out_specs=[pl.BlockSpec((1, T, D), lambda b: (b, 0, 0)),
                       pl.BlockSpec((1, T, 1), lambda b: (b, 0, 0))],
            out_shape=[jax.ShapeDtypeStruct((Bh, T, D), jnp.float32),
                       jax.ShapeDtypeStruct((Bh, T, 1), jnp.int32)],
        )(cc_half, x_half, w1b, b1r, w2b, b2r, gmr, btr)

    mesh = plsc.VectorSubcoreMesh(core_axis_name="c", subcore_axis_name="s")

    def decompress(seg_half, bidsg_half):
        n = bidsg_half.shape[0] * bidsg_half.shape[1]
        dec = functools.partial(
            pl.kernel,
            out_type=jax.ShapeDtypeStruct((n, D), jnp.float32),
            mesh=mesh,
            scratch_types=[
                pltpu.VMEM((_GCH,), jnp.int32),
                pltpu.VMEM((_GCH, D), jnp.float32),
                pltpu.SemaphoreType.DMA,
            ],
        )(_decompress_body)
        return dec(seg_half.reshape(n, D), bidsg_half.reshape(n))

    seg, bg = compress_ffn(hidden_states, cc3)
    return decompress(seg, bg).reshape(B, T, D)


# speedup vs baseline: 1.1919x; 1.0331x over previous
import functools

import jax
import jax.numpy as jnp
from jax import lax
from jax.experimental import pallas as pl
from jax.experimental.pallas import tpu as pltpu
from jax.experimental.pallas import tpu_sc as plsc

_B, _T, _D, _F = 8, 2048, 512, 2048
_TS = 256
_CK = 512
_NSC = 2
_NTEC = 16
_GCH = 64


def _fused_body(cc_ref, x_ref, w1_ref, b1_ref, w2_ref, b2_ref, gm_ref,
                bt_ref, o_ref, bidsg_ref, seg_s, cnt_s):
    cc = cc_ref[0]
    T = cc.shape[0]
    D = x_ref.shape[2]
    v = cc
    k = 1
    while k < T:
        z = jnp.zeros((k, 1), jnp.int32)
        v = v + jnp.concatenate([z, v[:-k]], axis=0)
        k *= 2
    bids = v - v[0:1]
    b = pl.program_id(0)
    bidsg_ref[0] = bids + b * T
    nseg = bids[T - 1, 0] + 1

    x = x_ref[0]
    xh = x.astype(jnp.bfloat16)
    ones_col = jnp.ones((_CK, 1), jnp.bfloat16)
    slot_iota = lax.broadcasted_iota(jnp.int32, (_CK, _TS), 1)
    cdims = (((0,), (0,)), ((), ()))

    for t in range(T // _TS):
        @pl.when(t * _TS < nseg)
        def _():
            seg_s[...] = jnp.zeros((_TS, D), jnp.float32)
            cnt_s[...] = jnp.zeros((_TS, 1), jnp.float32)
            for c in range(T // _CK):
                lo_id = bids[c * _CK, 0]
                hi_id = bids[c * _CK + _CK - 1, 0]
                @pl.when((hi_id >= t * _TS) & (lo_id < (t + 1) * _TS))
                def _():
                    bc = bids[c * _CK:(c + 1) * _CK]
                    oneh = (bc == slot_iota + t * _TS).astype(jnp.bfloat16)
                    cnt_s[...] += lax.dot_general(
                        oneh, ones_col, cdims,
                        preferred_element_type=jnp.float32)
                    seg_s[...] += lax.dot_general(
                        oneh, xh[c * _CK:(c + 1) * _CK], cdims,
                        preferred_element_type=jnp.float32)
            xm = seg_s[...] * (1.0 / jnp.maximum(cnt_s[...], 1.0))
            mh = xm.astype(jnp.bfloat16)
            p = jnp.dot(mh, w1_ref[...], preferred_element_type=jnp.float32)
            h1 = jnp.maximum(p + b1_ref[...], 0.0).astype(jnp.bfloat16)
            q = jnp.dot(h1, w2_ref[...], preferred_element_type=jnp.float32)
            acc = xm + q + b2_ref[...]
            mu = jnp.mean(acc, axis=-1, keepdims=True)
            d = acc - mu
            var = jnp.mean(d * d, axis=-1, keepdims=True)
            o_ref[0, pl.ds(t * _TS, _TS)] = (
                gm_ref[...] * d * lax.rsqrt(var + 1e-3) + bt_ref[...])


def _decompress_body(table_hbm, idxg_hbm, out_hbm, idx_v, rows_v, sem):
    c = lax.axis_index("c")
    w = lax.axis_index("s")
    wid = w * _NSC + c
    n_chunks = idxg_hbm.shape[0] // _GCH // (_NSC * _NTEC)
    for j in range(n_chunks):
        base = (wid * n_chunks + j) * _GCH
        pltpu.sync_copy(idxg_hbm.at[pl.ds(base, _GCH)], idx_v)
        pltpu.async_copy(table_hbm.at[idx_v], rows_v, sem).wait()
        pltpu.sync_copy(rows_v, out_hbm.at[pl.ds(base, _GCH)])


def kernel(hidden_states, chord_changes, W1, b1, W2, b2, gamma, beta):
    B, T, D = hidden_states.shape
    F = W1.shape[1]
    cc3 = chord_changes.reshape(B, T, 1)

    w1b = W1.astype(jnp.bfloat16)
    w2b = W2.astype(jnp.bfloat16)
    b1r = b1.reshape(1, F)
    b2r = b2.reshape(1, D)
    gmr = gamma.reshape(1, D)
    btr = beta.reshape(1, D)

    def compress_ffn(x_half, cc_half):
        Bh = x_half.shape[0]
        return pl.pallas_call(
            _fused_body,
            grid=(Bh,),
            in_specs=[
                pl.BlockSpec((1, T, 1), lambda b: (b, 0, 0)),
                pl.BlockSpec((1, T, D), lambda b: (b, 0, 0)),
                pl.BlockSpec((D, F), lambda b: (0, 0)),
                pl.BlockSpec((1, F), lambda b: (0, 0)),
                pl.BlockSpec((F, D), lambda b: (0, 0)),
                pl.BlockSpec((1, D), lambda b: (0, 0)),
                pl.BlockSpec((1, D), lambda b: (0, 0)),
                pl.BlockSpec((1, D), lambda b: (0, 0)),
            ],
            out_specs=[pl.BlockSpec((1, T, D), lambda b: (b, 0, 0)),
                       pl.BlockSpec((1, T, 1), lambda b: (b, 0, 0))],
            out_shape=[jax.ShapeDtypeStruct((Bh, T, D), jnp.float32),
                       jax.ShapeDtypeStruct((Bh, T, 1), jnp.int32)],
            scratch_shapes=[pltpu.VMEM((_TS, D), jnp.float32),
                            pltpu.VMEM((_TS, 1), jnp.float32)],
        )(cc_half, x_half, w1b, b1r, w2b, b2r, gmr, btr)

    mesh = plsc.VectorSubcoreMesh(core_axis_name="c", subcore_axis_name="s")

    def decompress(seg_half, bidsg_half):
        n = bidsg_half.shape[0] * bidsg_half.shape[1]
        dec = functools.partial(
            pl.kernel,
            out_type=jax.ShapeDtypeStruct((n, D), jnp.float32),
            mesh=mesh,
            scratch_types=[
                pltpu.VMEM((_GCH,), jnp.int32),
                pltpu.VMEM((_GCH, D), jnp.float32),
                pltpu.SemaphoreType.DMA,
            ],
        )(_decompress_body)
        return dec(seg_half.reshape(n, D), bidsg_half.reshape(n))

    seg, bg = compress_ffn(hidden_states, cc3)
    return decompress(seg, bg).reshape(B, T, D)
